# baseline (speedup 1.0000x reference)
import functools

import jax
import jax.numpy as jnp
from jax import lax
from jax.experimental import pallas as pl
from jax.experimental.pallas import tpu as pltpu
from jax.experimental.pallas import tpu_sc as plsc

B = 1024
L = 200
IN_G = 125
G_EMB = 64
R_EMB = 16
N_ROAD = 6
VOCAB = 100000
F_OUT = G_EMB + N_ROAD * R_EMB

ROWS = L * B
NW = 32

B_H = B // 2
ROWS_H = L * B_H
PER_W = ROWS_H // NW
R_CH = 128
N_CHUNKS = PER_W // R_CH

LB = 8
N_TC = L // LB


def _sc_roads_half(tables, idx_t, h):
    mesh = plsc.VectorSubcoreMesh(core_axis_name="c", subcore_axis_name="s")

    @functools.partial(
        pl.kernel,
        mesh=mesh,
        out_type=jax.ShapeDtypeStruct((L, B_H, 128), jnp.float32),
        scratch_types=[
            pltpu.VMEM((N_ROAD, R_CH), jnp.int32),
            pltpu.VMEM((N_ROAD, R_CH, R_EMB), jnp.float32),
            pltpu.SemaphoreType.DMA,
        ],
        compiler_params=pltpu.CompilerParams(use_tc_tiling_on_sc=False),
    )
    def k(t0_h, t1_h, t2_h, t3_h, t4_h, t5_h, idx_h, out_h, idx_v, rows_v, sem):
        tabs = [t0_h, t1_h, t2_h, t3_h, t4_h, t5_h]
        wid = lax.axis_index("s") * 2 + lax.axis_index("c")
        base = wid * PER_W

        def chunk_body(t, _):
            r0 = base + t * R_CH
            l0 = r0 // B_H
            b0 = r0 - l0 * B_H
            pltpu.sync_copy(
                idx_h.at[:, pl.ds(l0 * B + h * B_H + b0, R_CH)], idx_v.at[...]
            )

            descs = [
                pltpu.async_copy(
                    tabs[j].at[idx_v.at[j, :]],
                    rows_v.at[j],
                    sem,
                )
                for j in range(N_ROAD)
            ]
            for d in descs:
                d.wait()
            for j in range(N_ROAD):
                pltpu.sync_copy(
                    rows_v.at[j],
                    out_h.at[l0, pl.ds(b0, R_CH), pl.ds(j * R_EMB, R_EMB)],
                )
            return 0

        lax.fori_loop(0, N_CHUNKS, chunk_body, 0)

    return k(*tables, idx_t)


def _tc_assemble_half(x_t, w, b, roads_h, h, prev=None):
    def body(x_ref, w_ref, b_ref, r_ref, *rest):
        o_ref = rest[-1]
        x2 = x_ref[...].reshape(IN_G, LB * B_H)
        gps = lax.dot_general(
            w_ref[...], x2, (((0,), (0,)), ((), ())),
            preferred_element_type=jnp.float32,
        ) + b_ref[...]
        for l in range(LB):
            o_ref[l, 0:G_EMB, :] = gps[:, l * B_H:(l + 1) * B_H]
            o_ref[l, G_EMB:, :] = r_ref[l, :, 0:N_ROAD * R_EMB].T

    in_specs = [
        pl.BlockSpec((IN_G, LB, B_H), lambda i: (0, i, h)),
        pl.BlockSpec((IN_G, G_EMB), lambda i: (0, 0)),
        pl.BlockSpec((G_EMB, 1), lambda i: (0, 0)),
        pl.BlockSpec((LB, B_H, 128), lambda i: (i, 0, 0)),
    ]
    args = [x_t, w, b.reshape(G_EMB, 1), roads_h]
    aliases = {}
    if prev is not None:
        in_specs.append(pl.BlockSpec(memory_space=pltpu.MemorySpace.HBM))
        args.append(prev)
        aliases = {4: 0}

    return pl.pallas_call(
        body,
        grid=(N_TC,),
        in_specs=in_specs,
        out_specs=pl.BlockSpec((LB, F_OUT, B_H), lambda i: (i, 0, h)),
        out_shape=jax.ShapeDtypeStruct((L, F_OUT, B), jnp.float32),
        input_output_aliases=aliases,
    )(*args)


def kernel(stat_sensors, stat_gps, stat_road, W_sensors, b_sensors, W_geo, b_geo,
           emb_0, emb_1, emb_2, emb_3, emb_4, emb_5):
    tables = [emb_0, emb_1, emb_2, emb_3, emb_4, emb_5]
    idx_t = jnp.transpose(stat_road, (2, 1, 0)).reshape(N_ROAD, ROWS)
    roads0 = _sc_roads_half(tables, idx_t, 0)
    roads1 = _sc_roads_half(tables, idx_t, 1)
    x_t = jnp.transpose(stat_gps, (2, 1, 0))
    out = _tc_assemble_half(x_t, W_geo, b_geo, roads0, 0)
    out = _tc_assemble_half(x_t, W_geo, b_geo, roads1, 1, prev=out)
    return jnp.transpose(out, (2, 0, 1))


# speedup vs baseline: 1.0418x; 1.0418x over previous
import functools

import jax
import jax.numpy as jnp
from jax import lax
from jax.experimental import pallas as pl
from jax.experimental.pallas import tpu as pltpu
from jax.experimental.pallas import tpu_sc as plsc

B = 1024
L = 200
IN_G = 125
G_EMB = 64
R_EMB = 16
N_ROAD = 6
VOCAB = 100000
F_OUT = G_EMB + N_ROAD * R_EMB

ROWS = L * B
NW = 32
PER_W = ROWS // NW
R_CH = 256
N_CHUNKS = PER_W // R_CH

LB = 8


def _sc_roads(tables, idx_t):
    mesh = plsc.VectorSubcoreMesh(core_axis_name="c", subcore_axis_name="s")

    @functools.partial(
        pl.kernel,
        mesh=mesh,
        out_type=jax.ShapeDtypeStruct((L, B, 128), jnp.float32),
        scratch_types=[
            pltpu.VMEM((N_ROAD, R_CH), jnp.int32),
            pltpu.VMEM((N_ROAD, R_CH, R_EMB), jnp.float32),
            pltpu.SemaphoreType.DMA,
        ],
        compiler_params=pltpu.CompilerParams(use_tc_tiling_on_sc=False),
    )
    def k(t0_h, t1_h, t2_h, t3_h, t4_h, t5_h, idx_h, out_h, idx_v, rows_v, sem):
        tabs = [t0_h, t1_h, t2_h, t3_h, t4_h, t5_h]
        wid = lax.axis_index("s") * 2 + lax.axis_index("c")
        base = wid * PER_W

        def chunk_body(t, _):
            r0 = base + t * R_CH
            l0 = r0 // B
            b0 = r0 - l0 * B
            pltpu.sync_copy(idx_h.at[:, pl.ds(r0, R_CH)], idx_v.at[...])

            descs = [
                pltpu.async_copy(
                    tabs[j].at[idx_v.at[j, :]],
                    rows_v.at[j],
                    sem,
                )
                for j in range(N_ROAD)
            ]
            for d in descs:
                d.wait()
            for j in range(N_ROAD):
                pltpu.sync_copy(
                    rows_v.at[j],
                    out_h.at[l0, pl.ds(b0, R_CH), pl.ds(j * R_EMB, R_EMB)],
                )
            return 0

        lax.fori_loop(0, N_CHUNKS, chunk_body, 0)

    return k(*tables, idx_t)


def _tc_assemble(x_t, w, b, roads):
    def body(x_ref, w_ref, b_ref, r_ref, o_ref):
        x2 = x_ref[...].reshape(IN_G, LB * B)
        gps = lax.dot_general(
            w_ref[...], x2, (((0,), (0,)), ((), ())),
            preferred_element_type=jnp.float32,
        ) + b_ref[...]
        for l in range(LB):
            o_ref[l, 0:G_EMB, :] = gps[:, l * B:(l + 1) * B]
            o_ref[l, G_EMB:, :] = r_ref[l, :, 0:N_ROAD * R_EMB].T

    return pl.pallas_call(
        body,
        grid=(L // LB,),
        in_specs=[
            pl.BlockSpec((IN_G, LB, B), lambda i: (0, i, 0)),
            pl.BlockSpec((IN_G, G_EMB), lambda i: (0, 0)),
            pl.BlockSpec((G_EMB, 1), lambda i: (0, 0)),
            pl.BlockSpec((LB, B, 128), lambda i: (i, 0, 0)),
        ],
        out_specs=pl.BlockSpec((LB, F_OUT, B), lambda i: (i, 0, 0)),
        out_shape=jax.ShapeDtypeStruct((L, F_OUT, B), jnp.float32),
    )(x_t, w, b.reshape(G_EMB, 1), roads)


def kernel(stat_sensors, stat_gps, stat_road, W_sensors, b_sensors, W_geo, b_geo,
           emb_0, emb_1, emb_2, emb_3, emb_4, emb_5):
    tables = [emb_0, emb_1, emb_2, emb_3, emb_4, emb_5]
    idx_t = jnp.transpose(stat_road, (2, 1, 0)).reshape(N_ROAD, ROWS)
    roads = _sc_roads(tables, idx_t)
    x_t = jnp.transpose(stat_gps, (2, 1, 0))
    out = _tc_assemble(x_t, W_geo, b_geo, roads)
    return jnp.transpose(out, (2, 0, 1))
